# Initial kernel scaffold; baseline (speedup 1.0000x reference)
import jax
import jax.numpy as jnp
from jax.experimental import pallas as pl


def kernel(vertices, indices):
    raise NotImplementedError("write your pallas kernel here")


# speedup vs baseline: 10.2679x; 10.2679x over previous
import functools

import jax
import jax.numpy as jnp
from jax import lax
from jax.experimental import pallas as pl
from jax.experimental.pallas import tpu as pltpu
from jax.experimental.pallas import tpu_sc as plsc

NV = 100000
NT = 3200000
NVPAD = 100096
W = 32
IDS = NT * 4
IDSPW = IDS // W
CH = 3200
NCH = IDSPW // CH
GR = CH // 64


def _sc_body(xs, ys, zs, ihbm, pg, xg, yg, zg, tab, idxb, xb, yb, zb, sem):
    s = lax.axis_index("s")
    c = lax.axis_index("c")
    wid = s * 2 + c
    id0 = wid * IDSPW
    iota = lax.broadcasted_iota(jnp.int32, (16,), 0)

    for tabsrc, outdst in ((xs, xg), (ys, yg), (zs, zg)):
        pltpu.sync_copy(tabsrc, tab.at[pl.ds(0, NV)])

        def achunk(i, carry):
            base = id0 + i * CH
            pltpu.sync_copy(ihbm.at[pl.ds(base, CH)], idxb)

            def agrp(j, carry2):
                ids = idxb[pl.ds(j * 16, 16)]
                xb[pl.ds(j * 16, 16)] = plsc.load_gather(tab, [ids])
                return carry2

            lax.fori_loop(0, CH // 16, agrp, 0)
            pltpu.sync_copy(xb, outdst.at[pl.ds(base, CH)])
            return carry

        lax.fori_loop(0, NCH, achunk, 0)

    def zero_body(i, carry):
        tab[pl.ds(i * 16, 16)] = jnp.zeros((16,), jnp.float32)
        return carry

    lax.fori_loop(0, NVPAD // 16, zero_body, 0)

    def bchunk(i, carry):
        base = id0 + i * CH
        pltpu.sync_copy(ihbm.at[pl.ds(base, CH)], idxb)
        pltpu.sync_copy(xg.at[pl.ds(base, CH)], xb)
        pltpu.sync_copy(yg.at[pl.ds(base, CH)], yb)
        pltpu.sync_copy(zg.at[pl.ds(base, CH)], zb)

        def bgrp(j, carry2):
            jb = j * 64
            v = []
            for k in range(4):
                pos = jb + 4 * iota + k
                v.append((plsc.load_gather(xb, [pos]),
                          plsc.load_gather(yb, [pos]),
                          plsc.load_gather(zb, [pos])))
            e1 = [v[1][d] - v[0][d] for d in range(3)]
            e2 = [v[2][d] - v[0][d] for d in range(3)]
            e3 = [v[3][d] - v[0][d] for d in range(3)]
            det = (e1[0] * (e2[1] * e3[2] - e2[2] * e3[1])
                   - e1[1] * (e2[0] * e3[2] - e2[2] * e3[0])
                   + e1[2] * (e2[0] * e3[1] - e2[1] * e3[0]))
            d16 = jnp.abs(det) * jnp.float32(1.0 / 6.0)
            for k in range(4):
                vids = plsc.load_gather(idxb, [jb + 4 * iota + k])

                def wcond(p):
                    return jnp.max(p) > 0

                def wbody(p):
                    m = p > 0
                    cur = plsc.load_gather(tab, [vids])
                    new = jnp.maximum(cur, d16)
                    plsc.store_scatter(tab, [vids], new, mask=m)
                    chk = plsc.load_gather(tab, [vids])
                    return (m & (chk < d16)).astype(jnp.int32)

                lax.while_loop(wcond, wbody, jnp.ones((16,), jnp.int32))
            return carry2

        lax.fori_loop(0, GR, bgrp, 0)
        return carry

    lax.fori_loop(0, NCH, bchunk, 0)
    pltpu.sync_copy(tab, pg.at[pl.ds(wid * NVPAD, NVPAD)])


def _tc_reduce_body(p_ref, o_ref):
    o_ref[...] = jnp.max(p_ref[...], axis=0)


def kernel(vertices, indices):
    idx_flat = indices.astype(jnp.int32).reshape(IDS)
    xs = vertices[:, 0]
    ys = vertices[:, 1]
    zs = vertices[:, 2]
    mesh = plsc.VectorSubcoreMesh(core_axis_name="c", subcore_axis_name="s")
    partials, _, _, _ = functools.partial(
        pl.kernel,
        mesh=mesh,
        out_type=(
            jax.ShapeDtypeStruct((W * NVPAD,), jnp.float32),
            jax.ShapeDtypeStruct((IDS,), jnp.float32),
            jax.ShapeDtypeStruct((IDS,), jnp.float32),
            jax.ShapeDtypeStruct((IDS,), jnp.float32),
        ),
        scratch_types=[
            pltpu.VMEM((NVPAD,), jnp.float32),
            pltpu.VMEM((CH,), jnp.int32),
            pltpu.VMEM((CH,), jnp.float32),
            pltpu.VMEM((CH,), jnp.float32),
            pltpu.VMEM((CH,), jnp.float32),
            pltpu.SemaphoreType.DMA,
        ],
        compiler_params=pltpu.CompilerParams(
            needs_layout_passes=False, use_tc_tiling_on_sc=False),
    )(_sc_body)(xs, ys, zs, idx_flat)
    dens = pl.pallas_call(
        _tc_reduce_body,
        out_shape=jax.ShapeDtypeStruct((NVPAD,), jnp.float32),
    )(partials.reshape(W, NVPAD))
    return dens[:NV]


# speedup vs baseline: 10.8426x; 1.0560x over previous
import functools

import jax
import jax.numpy as jnp
from jax import lax
from jax.experimental import pallas as pl
from jax.experimental.pallas import tpu as pltpu
from jax.experimental.pallas import tpu_sc as plsc

NV = 100000
NT = 3200000
NVPAD = 100096
W = 32
IDS = NT * 4
IDSPW = IDS // W
CH = 3200
NCH = IDSPW // CH
GR = CH // 64
CHA = 8000
NCHA = IDSPW // CHA


def _sc_body(xs, ys, zs, ihbm, pg, xg, yg, zg,
             tab, idxa, xa, idxb, xb, yb, zb, sem):
    s = lax.axis_index("s")
    c = lax.axis_index("c")
    wid = s * 2 + c
    id0 = wid * IDSPW
    iota = lax.broadcasted_iota(jnp.int32, (16,), 0)

    for tabsrc, outdst in ((xs, xg), (ys, yg), (zs, zg)):
        pltpu.sync_copy(tabsrc, tab.at[pl.ds(0, NV)])

        def achunk(i, carry):
            base = id0 + i * CHA
            pltpu.sync_copy(ihbm.at[pl.ds(base, CHA)], idxa)

            def agrp(j, carry2):
                ids = idxa[pl.ds(j * 16, 16)]
                xa[pl.ds(j * 16, 16)] = plsc.load_gather(tab, [ids])
                return carry2

            lax.fori_loop(0, CHA // 16, agrp, 0)
            pltpu.sync_copy(xa, outdst.at[pl.ds(base, CHA)])
            return carry

        lax.fori_loop(0, NCHA, achunk, 0)

    def zero_body(i, carry):
        tab[pl.ds(i * 16, 16)] = jnp.zeros((16,), jnp.float32)
        return carry

    lax.fori_loop(0, NVPAD // 16, zero_body, 0)

    def bchunk(i, carry):
        base = id0 + i * CH
        h0 = pltpu.async_copy(ihbm.at[pl.ds(base, CH)], idxb, sem)
        h1 = pltpu.async_copy(xg.at[pl.ds(base, CH)], xb, sem)
        h2 = pltpu.async_copy(yg.at[pl.ds(base, CH)], yb, sem)
        h3 = pltpu.async_copy(zg.at[pl.ds(base, CH)], zb, sem)
        h0.wait()
        h1.wait()
        h2.wait()
        h3.wait()

        def bgrp(j, carry2):
            jb = j * 64
            v = []
            for k in range(4):
                pos = jb + 4 * iota + k
                v.append((plsc.load_gather(xb, [pos]),
                          plsc.load_gather(yb, [pos]),
                          plsc.load_gather(zb, [pos])))
            e1 = [v[1][d] - v[0][d] for d in range(3)]
            e2 = [v[2][d] - v[0][d] for d in range(3)]
            e3 = [v[3][d] - v[0][d] for d in range(3)]
            det = (e1[0] * (e2[1] * e3[2] - e2[2] * e3[1])
                   - e1[1] * (e2[0] * e3[2] - e2[2] * e3[0])
                   + e1[2] * (e2[0] * e3[1] - e2[1] * e3[0]))
            d16 = jnp.abs(det) * jnp.float32(1.0 / 6.0)
            for k in range(4):
                vids = plsc.load_gather(idxb, [jb + 4 * iota + k])

                def wcond(p):
                    return jnp.max(p) > 0

                def wbody(p):
                    m = p > 0
                    cur = plsc.load_gather(tab, [vids])
                    new = jnp.maximum(cur, d16)
                    plsc.store_scatter(tab, [vids], new, mask=m)
                    chk = plsc.load_gather(tab, [vids])
                    return (m & (chk < d16)).astype(jnp.int32)

                lax.while_loop(wcond, wbody, jnp.ones((16,), jnp.int32))
            return carry2

        lax.fori_loop(0, GR, bgrp, 0)
        return carry

    lax.fori_loop(0, NCH, bchunk, 0)
    pltpu.sync_copy(tab, pg.at[pl.ds(wid * NVPAD, NVPAD)])


def _tc_reduce_body(p_ref, o_ref):
    o_ref[...] = jnp.max(p_ref[...], axis=0)


def kernel(vertices, indices):
    idx_flat = indices.astype(jnp.int32).reshape(IDS)
    xs = vertices[:, 0]
    ys = vertices[:, 1]
    zs = vertices[:, 2]
    mesh = plsc.VectorSubcoreMesh(core_axis_name="c", subcore_axis_name="s")
    partials, _, _, _ = functools.partial(
        pl.kernel,
        mesh=mesh,
        out_type=(
            jax.ShapeDtypeStruct((W * NVPAD,), jnp.float32),
            jax.ShapeDtypeStruct((IDS,), jnp.float32),
            jax.ShapeDtypeStruct((IDS,), jnp.float32),
            jax.ShapeDtypeStruct((IDS,), jnp.float32),
        ),
        scratch_types=[
            pltpu.VMEM((NVPAD,), jnp.float32),
            pltpu.VMEM((CHA,), jnp.int32),
            pltpu.VMEM((CHA,), jnp.float32),
            pltpu.VMEM((CH,), jnp.int32),
            pltpu.VMEM((CH,), jnp.float32),
            pltpu.VMEM((CH,), jnp.float32),
            pltpu.VMEM((CH,), jnp.float32),
            pltpu.SemaphoreType.DMA,
        ],
        compiler_params=pltpu.CompilerParams(
            needs_layout_passes=False, use_tc_tiling_on_sc=False),
    )(_sc_body)(xs, ys, zs, idx_flat)
    dens = pl.pallas_call(
        _tc_reduce_body,
        out_shape=jax.ShapeDtypeStruct((NVPAD,), jnp.float32),
    )(partials.reshape(W, NVPAD))
    return dens[:NV]


# speedup vs baseline: 12.0998x; 1.1159x over previous
import functools

import jax
import jax.numpy as jnp
from jax import lax
from jax.experimental import pallas as pl
from jax.experimental.pallas import tpu as pltpu
from jax.experimental.pallas import tpu_sc as plsc

NV = 100000
NT = 3200000
NVPAD = 100096
W = 32
IDS = NT * 4
IDSPW = IDS // W
CH = 3200
NCH = IDSPW // CH
GR = CH // 64
CHA = 8000
NCHA = IDSPW // CHA


def _sc_body(xs, ys, zs, ihbm, pg, xg, yg, zg,
             tab, idxa, xa, idxb, xb, yb, zb, sem):
    s = lax.axis_index("s")
    c = lax.axis_index("c")
    wid = s * 2 + c
    id0 = wid * IDSPW
    iota = lax.broadcasted_iota(jnp.int32, (16,), 0)

    for tabsrc, outdst in ((xs, xg), (ys, yg), (zs, zg)):
        pltpu.sync_copy(tabsrc, tab.at[pl.ds(0, NV)])

        def achunk(i, carry):
            base = id0 + i * CHA
            pltpu.sync_copy(ihbm.at[pl.ds(base, CHA)], idxa)

            def agrp(j, carry2):
                for u in range(4):
                    ids = idxa[pl.ds(j * 64 + u * 16, 16)]
                    xa[pl.ds(j * 64 + u * 16, 16)] = plsc.load_gather(tab, [ids])
                return carry2

            lax.fori_loop(0, CHA // 64, agrp, 0)
            pltpu.sync_copy(xa, outdst.at[pl.ds(base, CHA)])
            return carry

        lax.fori_loop(0, NCHA, achunk, 0)

    def zero_body(i, carry):
        tab[pl.ds(i * 16, 16)] = jnp.zeros((16,), jnp.float32)
        return carry

    lax.fori_loop(0, NVPAD // 16, zero_body, 0)

    def bchunk(i, carry):
        base = id0 + i * CH
        h0 = pltpu.async_copy(ihbm.at[pl.ds(base, CH)], idxb, sem)
        h1 = pltpu.async_copy(xg.at[pl.ds(base, CH)], xb, sem)
        h2 = pltpu.async_copy(yg.at[pl.ds(base, CH)], yb, sem)
        h3 = pltpu.async_copy(zg.at[pl.ds(base, CH)], zb, sem)
        h0.wait()
        h1.wait()
        h2.wait()
        h3.wait()

        def bgrp(j, carry2):
            jb = j * 64
            v = []
            for k in range(4):
                pos = jb + 4 * iota + k
                v.append((plsc.load_gather(xb, [pos]),
                          plsc.load_gather(yb, [pos]),
                          plsc.load_gather(zb, [pos])))
            e1 = [v[1][d] - v[0][d] for d in range(3)]
            e2 = [v[2][d] - v[0][d] for d in range(3)]
            e3 = [v[3][d] - v[0][d] for d in range(3)]
            det = (e1[0] * (e2[1] * e3[2] - e2[2] * e3[1])
                   - e1[1] * (e2[0] * e3[2] - e2[2] * e3[0])
                   + e1[2] * (e2[0] * e3[1] - e2[1] * e3[0]))
            d16 = jnp.abs(det) * jnp.float32(1.0 / 6.0)
            vids = [plsc.load_gather(idxb, [jb + 4 * iota + k])
                    for k in range(4)]

            def wcond(ps):
                p0, p1, p2, p3 = ps
                return jnp.max(p0 | p1 | p2 | p3) > 0

            def wbody(ps):
                for k in range(4):
                    m = ps[k] > 0
                    cur = plsc.load_gather(tab, [vids[k]])
                    new = jnp.maximum(cur, d16)
                    plsc.store_scatter(tab, [vids[k]], new, mask=m)
                outs = []
                for k in range(4):
                    chk = plsc.load_gather(tab, [vids[k]])
                    outs.append(((ps[k] > 0) & (chk < d16)).astype(jnp.int32))
                return tuple(outs)

            ones = jnp.ones((16,), jnp.int32)
            lax.while_loop(wcond, wbody, (ones, ones, ones, ones))
            return carry2

        lax.fori_loop(0, GR, bgrp, 0)
        return carry

    lax.fori_loop(0, NCH, bchunk, 0)
    pltpu.sync_copy(tab, pg.at[pl.ds(wid * NVPAD, NVPAD)])


def _tc_reduce_body(p_ref, o_ref):
    o_ref[...] = jnp.max(p_ref[...], axis=0)


def kernel(vertices, indices):
    idx_flat = indices.astype(jnp.int32).reshape(IDS)
    xs = vertices[:, 0]
    ys = vertices[:, 1]
    zs = vertices[:, 2]
    mesh = plsc.VectorSubcoreMesh(core_axis_name="c", subcore_axis_name="s")
    partials, _, _, _ = functools.partial(
        pl.kernel,
        mesh=mesh,
        out_type=(
            jax.ShapeDtypeStruct((W * NVPAD,), jnp.float32),
            jax.ShapeDtypeStruct((IDS,), jnp.float32),
            jax.ShapeDtypeStruct((IDS,), jnp.float32),
            jax.ShapeDtypeStruct((IDS,), jnp.float32),
        ),
        scratch_types=[
            pltpu.VMEM((NVPAD,), jnp.float32),
            pltpu.VMEM((CHA,), jnp.int32),
            pltpu.VMEM((CHA,), jnp.float32),
            pltpu.VMEM((CH,), jnp.int32),
            pltpu.VMEM((CH,), jnp.float32),
            pltpu.VMEM((CH,), jnp.float32),
            pltpu.VMEM((CH,), jnp.float32),
            pltpu.SemaphoreType.DMA,
        ],
        compiler_params=pltpu.CompilerParams(
            needs_layout_passes=False, use_tc_tiling_on_sc=False),
    )(_sc_body)(xs, ys, zs, idx_flat)
    dens = pl.pallas_call(
        _tc_reduce_body,
        out_shape=jax.ShapeDtypeStruct((NVPAD,), jnp.float32),
    )(partials.reshape(W, NVPAD))
    return dens[:NV]


# speedup vs baseline: 12.2399x; 1.0116x over previous
import functools

import jax
import jax.numpy as jnp
from jax import lax
from jax.experimental import pallas as pl
from jax.experimental.pallas import tpu as pltpu
from jax.experimental.pallas import tpu_sc as plsc

NV = 100000
NT = 3200000
NVPAD = 100096
W = 32
IDS = NT * 4
IDSPW = IDS // W
CH = 3200
NCH = IDSPW // CH
GR = CH // 64
CHA = 8000
NCHA = IDSPW // CHA


def _sc_body(xs, ys, zs, ihbm, pg, xg, yg, zg,
             tab, idxa, xa, idxb, xb, yb, zb, sem):
    s = lax.axis_index("s")
    c = lax.axis_index("c")
    wid = s * 2 + c
    id0 = wid * IDSPW
    iota = lax.broadcasted_iota(jnp.int32, (16,), 0)

    for tabsrc, outdst in ((xs, xg), (ys, yg), (zs, zg)):
        pltpu.sync_copy(tabsrc, tab.at[pl.ds(0, NV)])

        def achunk(i, carry):
            base = id0 + i * CHA
            pltpu.sync_copy(ihbm.at[pl.ds(base, CHA)], idxa)

            def agrp(j, carry2):
                for u in range(4):
                    ids = idxa[pl.ds(j * 64 + u * 16, 16)]
                    xa[pl.ds(j * 64 + u * 16, 16)] = plsc.load_gather(tab, [ids])
                return carry2

            lax.fori_loop(0, CHA // 64, agrp, 0)
            pltpu.sync_copy(xa, outdst.at[pl.ds(base, CHA)])
            return carry

        lax.fori_loop(0, NCHA, achunk, 0)

    def zero_body(i, carry):
        tab[pl.ds(i * 16, 16)] = jnp.zeros((16,), jnp.float32)
        return carry

    lax.fori_loop(0, NVPAD // 16, zero_body, 0)

    def bchunk(i, carry):
        base = id0 + i * CH
        h0 = pltpu.async_copy(ihbm.at[pl.ds(base, CH)], idxb, sem)
        h1 = pltpu.async_copy(xg.at[pl.ds(base, CH)], xb, sem)
        h2 = pltpu.async_copy(yg.at[pl.ds(base, CH)], yb, sem)
        h3 = pltpu.async_copy(zg.at[pl.ds(base, CH)], zb, sem)
        h0.wait()
        h1.wait()
        h2.wait()
        h3.wait()

        def bgrp(j, carry2):
            jb = j * 64
            v = []
            for k in range(4):
                pos = jb + 4 * iota + k
                v.append((plsc.load_gather(xb, [pos]),
                          plsc.load_gather(yb, [pos]),
                          plsc.load_gather(zb, [pos])))
            e1 = [v[1][d] - v[0][d] for d in range(3)]
            e2 = [v[2][d] - v[0][d] for d in range(3)]
            e3 = [v[3][d] - v[0][d] for d in range(3)]
            det = (e1[0] * (e2[1] * e3[2] - e2[2] * e3[1])
                   - e1[1] * (e2[0] * e3[2] - e2[2] * e3[0])
                   + e1[2] * (e2[0] * e3[1] - e2[1] * e3[0]))
            d16 = jnp.abs(det) * jnp.float32(1.0 / 6.0)
            vids = [plsc.load_gather(idxb, [jb + 4 * iota + k])
                    for k in range(4)]
            for k in range(4):
                cur = plsc.load_gather(tab, [vids[k]])
                plsc.store_scatter(tab, [vids[k]], jnp.maximum(cur, d16))
            pend0 = []
            for k in range(4):
                chk = plsc.load_gather(tab, [vids[k]])
                pend0.append((chk < d16).astype(jnp.int32))

            def wcond(ps):
                p0, p1, p2, p3 = ps
                return jnp.max(p0 | p1 | p2 | p3) > 0

            def wbody(ps):
                for k in range(4):
                    m = ps[k] > 0
                    cur = plsc.load_gather(tab, [vids[k]])
                    new = jnp.maximum(cur, d16)
                    plsc.store_scatter(tab, [vids[k]], new, mask=m)
                outs = []
                for k in range(4):
                    chk = plsc.load_gather(tab, [vids[k]])
                    outs.append(((ps[k] > 0) & (chk < d16)).astype(jnp.int32))
                return tuple(outs)

            lax.while_loop(wcond, wbody, tuple(pend0))
            return carry2

        lax.fori_loop(0, GR, bgrp, 0)
        return carry

    lax.fori_loop(0, NCH, bchunk, 0)
    pltpu.sync_copy(tab, pg.at[pl.ds(wid * NVPAD, NVPAD)])


def _tc_reduce_body(p_ref, o_ref):
    o_ref[...] = jnp.max(p_ref[...], axis=0)


def kernel(vertices, indices):
    idx_flat = indices.astype(jnp.int32).reshape(IDS)
    xs = vertices[:, 0]
    ys = vertices[:, 1]
    zs = vertices[:, 2]
    mesh = plsc.VectorSubcoreMesh(core_axis_name="c", subcore_axis_name="s")
    partials, _, _, _ = functools.partial(
        pl.kernel,
        mesh=mesh,
        out_type=(
            jax.ShapeDtypeStruct((W * NVPAD,), jnp.float32),
            jax.ShapeDtypeStruct((IDS,), jnp.float32),
            jax.ShapeDtypeStruct((IDS,), jnp.float32),
            jax.ShapeDtypeStruct((IDS,), jnp.float32),
        ),
        scratch_types=[
            pltpu.VMEM((NVPAD,), jnp.float32),
            pltpu.VMEM((CHA,), jnp.int32),
            pltpu.VMEM((CHA,), jnp.float32),
            pltpu.VMEM((CH,), jnp.int32),
            pltpu.VMEM((CH,), jnp.float32),
            pltpu.VMEM((CH,), jnp.float32),
            pltpu.VMEM((CH,), jnp.float32),
            pltpu.SemaphoreType.DMA,
        ],
        compiler_params=pltpu.CompilerParams(
            needs_layout_passes=False, use_tc_tiling_on_sc=False),
    )(_sc_body)(xs, ys, zs, idx_flat)
    dens = pl.pallas_call(
        _tc_reduce_body,
        out_shape=jax.ShapeDtypeStruct((NVPAD,), jnp.float32),
    )(partials.reshape(W, NVPAD))
    return dens[:NV]
